# baseline (speedup 1.0000x reference)
import functools

import jax
import jax.numpy as jnp
from jax import lax
from jax.experimental import pallas as pl
from jax.experimental.pallas import tpu as pltpu
from jax.experimental.pallas import tpu_sc as plsc

VOCAB = 6
NAUX = 5
HID = 512
NC = 2
NS = 16
NW = NC * NS
CH = 32


def _body(ids_hbm, aux_hbm, zeros_hbm, out_hbm,
          idsall, auxall, buf0, buf1, semo0, semo1):
    n = out_hbm.shape[0]
    rows_per_w = n // NW
    nchunk = rows_per_w // CH
    npair = nchunk // 2
    seq = aux_hbm.shape[1] // NAUX
    wid = lax.axis_index("s") * NC + lax.axis_index("c")
    base0 = wid * rows_per_w
    ib = base0 // seq
    sbase = base0 - ib * seq

    iota = lax.iota(jnp.int32, 16)
    ones = jnp.ones((16,), jnp.float32)
    zero16 = jnp.zeros((16,), jnp.float32)
    colidx = jnp.clip(iota - VOCAB, 0, NAUX - 1)
    auxmask = (iota >= VOCAB) & (iota < VOCAB + NAUX)

    def fill(buf, c):
        off = c * CH
        for row in range(CH):
            avidx = jnp.full((16,), (off + row) * NAUX, jnp.int32) + colidx
            av = plsc.load_gather(auxall, [avidx])
            buf[row, 0:16] = jnp.where(auxmask, av, zero16)
        for g in range(CH // 16):
            rows16 = off + g * 16 + iota
            idsv = plsc.load_gather(idsall, [rows16])
            plsc.store_scatter(buf, [g * 16 + iota, idsv], ones)

    def out_start(buf, c, semo):
        pltpu.async_copy(buf, out_hbm.at[pl.ds(base0 + c * CH, CH)], semo)

    def out_wait(buf, semo):
        pltpu.make_async_copy(buf, out_hbm.at[pl.ds(base0, CH)], semo).wait()

    pltpu.sync_copy(ids_hbm.at[pl.ds(base0, rows_per_w)], idsall)
    pltpu.sync_copy(aux_hbm.at[ib, pl.ds(sbase * NAUX, rows_per_w * NAUX)], auxall)
    pltpu.sync_copy(zeros_hbm, buf0)
    pltpu.sync_copy(zeros_hbm, buf1)

    fill(buf0, 0)
    out_start(buf0, 0, semo0)
    fill(buf1, 1)
    out_start(buf1, 1, semo1)

    def pair(p, carry):
        c0 = 2 * p
        out_wait(buf0, semo0)
        fill(buf0, c0)
        out_start(buf0, c0, semo0)
        out_wait(buf1, semo1)
        fill(buf1, c0 + 1)
        out_start(buf1, c0 + 1, semo1)
        return carry

    lax.fori_loop(1, npair, pair, 0)
    out_wait(buf0, semo0)
    out_wait(buf1, semo1)


def kernel(input_ids, aux_features):
    B, S = input_ids.shape
    N = B * S
    rows_per_w = N // NW
    aux2d = aux_features.reshape(B, S * NAUX)
    zeros = jnp.zeros((CH, HID), jnp.float32)

    k = functools.partial(
        pl.kernel,
        out_type=jax.ShapeDtypeStruct((N, HID), jnp.float32),
        mesh=plsc.VectorSubcoreMesh(core_axis_name="c", subcore_axis_name="s"),
        compiler_params=pltpu.CompilerParams(
            needs_layout_passes=False, use_tc_tiling_on_sc=False),
        scratch_types=[
            pltpu.VMEM((rows_per_w,), jnp.int32),
            pltpu.VMEM((rows_per_w * NAUX,), jnp.float32),
            pltpu.VMEM((CH, HID), jnp.float32),
            pltpu.VMEM((CH, HID), jnp.float32),
            pltpu.SemaphoreType.DMA,
            pltpu.SemaphoreType.DMA,
        ],
    )(_body)
    out = k(input_ids.reshape(N).astype(jnp.int32), aux2d, zeros)
    return out.reshape(B, S, HID)


# speedup vs baseline: 1.9770x; 1.9770x over previous
import functools

import jax
import jax.numpy as jnp
from jax import lax
from jax.experimental import pallas as pl
from jax.experimental.pallas import tpu as pltpu
from jax.experimental.pallas import tpu_sc as plsc

VOCAB = 6
NAUX = 5
HID = 512
NC = 2
NS = 16
NW = NC * NS
CH = 32


def _body(ids_hbm, aux_hbm, zeros_hbm, out_hbm,
          idsall, auxall, buf0, buf1, semo0, semo1):
    n = out_hbm.shape[0]
    rows_per_w = n // NW
    nchunk = rows_per_w // CH
    npair = nchunk // 2
    seq = ids_hbm.shape[1]
    wid = lax.axis_index("s") * NC + lax.axis_index("c")
    base0 = wid * rows_per_w
    ib = base0 // seq
    sbase = base0 - ib * seq

    iota = lax.iota(jnp.int32, 16)
    ones = jnp.ones((16,), jnp.float32)
    zero16 = jnp.zeros((16,), jnp.float32)
    colidx = jnp.clip(iota - VOCAB, 0, NAUX - 1)
    auxmask = (iota >= VOCAB) & (iota < VOCAB + NAUX)

    def fill(buf, c):
        off = c * CH
        for row in range(CH):
            avidx = jnp.full((16,), (off + row) * NAUX, jnp.int32) + colidx
            av = plsc.load_gather(auxall, [avidx])
            buf[row, 0:16] = jnp.where(auxmask, av, zero16)
        for g in range(CH // 16):
            rows16 = off + g * 16 + iota
            idsv = plsc.load_gather(idsall, [rows16])
            plsc.store_scatter(buf, [g * 16 + iota, idsv], ones)

    def out_start(buf, c, semo):
        pltpu.async_copy(buf, out_hbm.at[pl.ds(base0 + c * CH, CH)], semo)

    def out_wait(buf, semo):
        pltpu.make_async_copy(buf, out_hbm.at[pl.ds(base0, CH)], semo).wait()

    pltpu.sync_copy(ids_hbm.at[ib, pl.ds(sbase, rows_per_w)], idsall)
    pltpu.sync_copy(aux_hbm.at[pl.ds(base0 * NAUX, rows_per_w * NAUX)], auxall)
    pltpu.sync_copy(zeros_hbm, buf0)
    pltpu.sync_copy(zeros_hbm, buf1)

    fill(buf0, 0)
    out_start(buf0, 0, semo0)
    fill(buf1, 1)
    out_start(buf1, 1, semo1)

    def pair(p, carry):
        c0 = 2 * p
        out_wait(buf0, semo0)
        fill(buf0, c0)
        out_start(buf0, c0, semo0)
        out_wait(buf1, semo1)
        fill(buf1, c0 + 1)
        out_start(buf1, c0 + 1, semo1)
        return carry

    lax.fori_loop(1, npair, pair, 0)
    out_wait(buf0, semo0)
    out_wait(buf1, semo1)


def kernel(input_ids, aux_features):
    B, S = input_ids.shape
    N = B * S
    rows_per_w = N // NW
    aux1 = aux_features.reshape(N * NAUX)
    zeros = jnp.zeros((CH, HID), jnp.float32)

    k = functools.partial(
        pl.kernel,
        out_type=jax.ShapeDtypeStruct((N, HID), jnp.float32),
        mesh=plsc.VectorSubcoreMesh(core_axis_name="c", subcore_axis_name="s"),
        compiler_params=pltpu.CompilerParams(
            needs_layout_passes=False, use_tc_tiling_on_sc=True),
        scratch_types=[
            pltpu.VMEM((rows_per_w,), jnp.int32),
            pltpu.VMEM((rows_per_w * NAUX,), jnp.float32),
            pltpu.VMEM((CH, HID), jnp.float32),
            pltpu.VMEM((CH, HID), jnp.float32),
            pltpu.SemaphoreType.DMA,
            pltpu.SemaphoreType.DMA,
        ],
    )(_body)
    out = k(input_ids.astype(jnp.int32), aux1, zeros)
    return out.reshape(B, S, HID)


# speedup vs baseline: 2.3538x; 1.1906x over previous
import functools

import jax
import jax.numpy as jnp
from jax import lax
from jax.experimental import pallas as pl
from jax.experimental.pallas import tpu as pltpu
from jax.experimental.pallas import tpu_sc as plsc

VOCAB = 6
NAUX = 5
HID = 512
NC = 2
NS = 16
NW = NC * NS
CH = 32


def _body(ids_hbm, aux_hbm, zeros_hbm, out_hbm,
          idsall, auxall, buf0, buf1, semo0, semo1):
    n = out_hbm.shape[0]
    rows_per_w = n // NW
    nchunk = rows_per_w // CH
    npair = nchunk // 2
    seq = ids_hbm.shape[1]
    wid = lax.axis_index("s") * NC + lax.axis_index("c")
    base0 = wid * rows_per_w
    ib = base0 // seq
    sbase = base0 - ib * seq

    iota = lax.iota(jnp.int32, 16)
    ones = jnp.ones((16,), jnp.float32)
    zero16 = jnp.zeros((16,), jnp.float32)
    colidx = jnp.clip(iota - VOCAB, 0, NAUX - 1)
    auxmask = (iota >= VOCAB) & (iota < VOCAB + NAUX)

    def fill(buf, c):
        off = c * CH
        for row in range(CH):
            avidx = jnp.full((16,), (off + row) * NAUX, jnp.int32) + colidx
            av = plsc.load_gather(auxall, [avidx])
            buf[row, 0:16] = jnp.where(auxmask, av, zero16)
        for g in range(CH // 16):
            rows16 = off + g * 16 + iota
            idsv = plsc.load_gather(idsall, [rows16])
            plsc.store_scatter(buf, [g * 16 + iota, idsv], ones)

    def out_start(buf, c, semo):
        pltpu.async_copy(buf, out_hbm.at[pl.ds(base0 + c * CH, CH)], semo)

    def out_wait(buf, semo):
        pltpu.make_async_copy(buf, out_hbm.at[pl.ds(base0, CH)], semo).wait()

    pltpu.sync_copy(ids_hbm.at[ib, pl.ds(sbase, rows_per_w)], idsall)
    pltpu.sync_copy(aux_hbm.at[ib, pl.ds(sbase * NAUX, rows_per_w * NAUX)], auxall)
    pltpu.sync_copy(zeros_hbm, buf0)
    pltpu.sync_copy(zeros_hbm, buf1)

    fill(buf0, 0)
    out_start(buf0, 0, semo0)
    fill(buf1, 1)
    out_start(buf1, 1, semo1)

    def pair(p, carry):
        c0 = 2 * p
        out_wait(buf0, semo0)
        fill(buf0, c0)
        out_start(buf0, c0, semo0)
        out_wait(buf1, semo1)
        fill(buf1, c0 + 1)
        out_start(buf1, c0 + 1, semo1)
        return carry

    lax.fori_loop(1, npair, pair, 0)
    out_wait(buf0, semo0)
    out_wait(buf1, semo1)


def kernel(input_ids, aux_features):
    B, S = input_ids.shape
    N = B * S
    rows_per_w = N // NW
    aux2d = aux_features.reshape(B, S * NAUX)
    zeros = jnp.zeros((CH, HID), jnp.float32)

    k = functools.partial(
        pl.kernel,
        out_type=jax.ShapeDtypeStruct((N, HID), jnp.float32),
        mesh=plsc.VectorSubcoreMesh(core_axis_name="c", subcore_axis_name="s"),
        compiler_params=pltpu.CompilerParams(
            needs_layout_passes=False, use_tc_tiling_on_sc=True),
        scratch_types=[
            pltpu.VMEM((rows_per_w,), jnp.int32),
            pltpu.VMEM((rows_per_w * NAUX,), jnp.float32),
            pltpu.VMEM((CH, HID), jnp.float32),
            pltpu.VMEM((CH, HID), jnp.float32),
            pltpu.SemaphoreType.DMA,
            pltpu.SemaphoreType.DMA,
        ],
    )(_body)
    out = k(input_ids.astype(jnp.int32), aux2d, zeros)
    return out.reshape(B, S, HID)
